# Initial kernel scaffold; baseline (speedup 1.0000x reference)
import jax
import jax.numpy as jnp
from jax.experimental import pallas as pl


def kernel(x, edge_index, W, attn_l, attn_r, bias):
    raise NotImplementedError("write your pallas kernel here")


# speedup vs baseline: 7.5339x; 7.5339x over previous
import jax
import jax.numpy as jnp
from jax import lax
from jax.experimental import pallas as pl
from jax.experimental.pallas import tpu as pltpu
from jax.experimental.pallas import tpu_sc as plsc

N = 10000
E = 320000
D = 128
H = 4
F = 32
HF = H * F

NC = 2
NS = 16
EW = E // NS
C = 80
NCHUNK = EW // C
NH = N // NC
NL = NH + 8
NT_STRIDE = 312

_I32 = jnp.int32
_F32 = jnp.float32


def _pro_body(x_ref, w_ref, a_ref, feat_ref, elr_ref):
    feat = jnp.dot(x_ref[...], w_ref[...], preferred_element_type=_F32)
    feat_ref[...] = feat
    elr_ref[...] = jnp.dot(feat, a_ref[...], preferred_element_type=_F32)


def _prologue(x, W, A):
    BR = 1000
    return pl.pallas_call(
        _pro_body,
        grid=(N // BR,),
        in_specs=[
            pl.BlockSpec((BR, D), lambda i: (i, 0)),
            pl.BlockSpec((D, HF), lambda i: (0, 0)),
            pl.BlockSpec((D, 16), lambda i: (0, 0)),
        ],
        out_specs=[
            pl.BlockSpec((BR, HF), lambda i: (i, 0)),
            pl.BlockSpec((BR, 16), lambda i: (i, 0)),
        ],
        out_shape=[
            jax.ShapeDtypeStruct((N, HF), _F32),
            jax.ShapeDtypeStruct((N, 16), _F32),
        ],
    )(x, W, A)


def _sc_body(elr_hbm, pk_hbm, feat_hbm,
             rst_out, den_out,
             pk_v0, src_v0, dst_v0, dstl_v0, rows_v0, ee_v0, els_v0, eld_v0,
             pk_v1, src_v1, dst_v1, dstl_v1, rows_v1, ee_v1, els_v1, eld_v1,
             rst_sh, den_sh, sem_g0, sem_e0, sem_s0, sem_g1, sem_e1, sem_s1):
    cid = lax.axis_index("c")
    sid = lax.axis_index("s")
    r0 = sid * NT_STRIDE

    iota = lax.iota(_I32, 16)
    ebase = sid * EW
    nlo = cid * NH

    bufs = [
        (pk_v0, src_v0, dst_v0, dstl_v0, rows_v0, ee_v0, els_v0, eld_v0,
         sem_g0, sem_e0, sem_s0),
        (pk_v1, src_v1, dst_v1, dstl_v1, rows_v1, ee_v1, els_v1, eld_v1,
         sem_g1, sem_e1, sem_s1),
    ]

    zv = jnp.zeros((16,), _F32)

    def zrow(g, c2):
        gv = jnp.full((16,), g * 16, _I32) + iota
        for k in range(HF):
            plsc.store_scatter(rows_v0, [gv, jnp.full((16,), k, _I32)], zv)
        for k in range(16):
            plsc.store_scatter(ee_v0, [gv, jnp.full((16,), k, _I32)], zv)
        return c2

    lax.fori_loop(0, C // 16, zrow, 0)
    for j in range(4):
        off = jnp.minimum((sid * 4 + j) * C, NL - C)
        pltpu.sync_copy(rows_v0, rst_sh.at[pl.ds(off, C)])
        pltpu.sync_copy(ee_v0, den_sh.at[pl.ds(off, C)])
    plsc.subcore_barrier()

    def process(i, b, first):
        (pk_v, src_v, dst_v, dstl_v, rows_v, ee_v, els_v, eld_v,
         sem_g, sem_e, sem_s) = bufs[b]
        base = ebase + i * C
        if not first:
            pltpu.make_async_copy(rows_v, rst_sh.at[dstl_v], sem_s).wait()
            pltpu.make_async_copy(ee_v, den_sh.at[dstl_v], sem_s).wait()
        pltpu.sync_copy(pk_hbm.at[pl.ds(base, C)], pk_v)

        def unpack(g, c2):
            gv = jnp.full((16,), g * 16, _I32) + iota
            p16 = plsc.load_gather(pk_v, [gv])
            s16 = p16 & 0x3FFF
            d16 = p16 >> 14
            plsc.store_scatter(src_v, [gv], s16)
            plsc.store_scatter(dst_v, [gv], d16)
            dl = d16 - nlo
            dl = jnp.where((dl >= 0) & (dl < NH), dl, NH)
            plsc.store_scatter(dstl_v, [gv], dl)
            return c2

        lax.fori_loop(0, C // 16, unpack, 0)
        cp = pltpu.async_copy(feat_hbm.at[src_v], rows_v, sem_g)
        ce1 = pltpu.async_copy(elr_hbm.at[src_v], els_v, sem_e)
        ce2 = pltpu.async_copy(elr_hbm.at[dst_v], eld_v, sem_e)
        ce1.wait()
        ce2.wait()

        def eeg(g, c2):
            gv = jnp.full((16,), g * 16, _I32) + iota
            for h in range(H):
                el = plsc.load_gather(els_v, [gv, jnp.full((16,), h, _I32)])
                er = plsc.load_gather(eld_v, [gv, jnp.full((16,), H + h, _I32)])
                e = el + er
                e = jnp.where(e >= 0.0, e, e * 0.2)
                ee = jnp.exp(e)
                plsc.store_scatter(ee_v, [gv, jnp.full((16,), h, _I32)], ee)
            return c2

        lax.fori_loop(0, C // 16, eeg, 0)
        cp.wait()

        def mulg(g, c2):
            gv = jnp.full((16,), g * 16, _I32) + iota
            for h in range(H):
                scale = plsc.load_gather(ee_v, [gv, jnp.full((16,), h, _I32)])
                for k in range(F):
                    fv = jnp.full((16,), h * F + k, _I32)
                    v = plsc.load_gather(rows_v, [gv, fv])
                    plsc.store_scatter(rows_v, [gv, fv], v * scale)
            return c2

        lax.fori_loop(0, C // 16, mulg, 0)

        pltpu.async_copy(rows_v, rst_sh.at[dstl_v], sem_s, add=True)
        pltpu.async_copy(ee_v, den_sh.at[dstl_v], sem_s, add=True)

    process(0, 0, True)
    process(1, 1, True)

    def it_body(j, carry):
        process(2 * j + 2, 0, False)
        process(2 * j + 3, 1, False)
        return carry

    lax.fori_loop(0, NCHUNK // 2 - 1, it_body, 0)
    for b in range(2):
        (pk_v, src_v, dst_v, dstl_v, rows_v, ee_v, els_v, eld_v,
         sem_g, sem_e, sem_s) = bufs[b]
        pltpu.make_async_copy(rows_v, rst_sh.at[dstl_v], sem_s).wait()
        pltpu.make_async_copy(ee_v, den_sh.at[dstl_v], sem_s).wait()
    plsc.subcore_barrier()

    pltpu.sync_copy(rst_sh.at[pl.ds(r0, 320)],
                    rst_out.at[pl.ds(nlo + r0, 320)])
    pltpu.sync_copy(den_sh.at[pl.ds(r0, 320)],
                    den_out.at[pl.ds(nlo + r0, 320)])


def _sc_main(elr, pk, feat):
    mesh = plsc.VectorSubcoreMesh(
        core_axis_name="c", subcore_axis_name="s",
        num_cores=NC, num_subcores=NS)
    fn = pl.kernel(
        _sc_body,
        out_type=[
            pltpu.MemorySpace.HBM((N, HF), _F32),
            pltpu.MemorySpace.HBM((N, 16), _F32),
        ],
        mesh=mesh,
        scratch_types=(
            [pltpu.VMEM((C,), _I32),
             pltpu.VMEM((C,), _I32),
             pltpu.VMEM((C,), _I32),
             pltpu.VMEM((C,), _I32),
             pltpu.VMEM((C, HF), _F32),
             pltpu.VMEM((C, 16), _F32),
             pltpu.VMEM((C, 16), _F32),
             pltpu.VMEM((C, 16), _F32)] * 2 +
            [pltpu.VMEM_SHARED((NL, HF), _F32),
             pltpu.VMEM_SHARED((NL, 16), _F32)] +
            [pltpu.SemaphoreType.DMA] * 6
        ),
        compiler_params=pltpu.CompilerParams(
            needs_layout_passes=False, use_tc_tiling_on_sc=False),
    )
    return fn(elr, pk, feat)


def _epi_body(r_ref, d_ref, b_ref, out_ref):
    d4 = d_ref[...][:, :H]
    recip = jnp.where(d4 > 0.0, 1.0 / d4, 0.0)
    col = lax.broadcasted_iota(_I32, (H, HF), 1)
    row = lax.broadcasted_iota(_I32, (H, HF), 0)
    expand = (col // F == row).astype(_F32)
    scale = jnp.dot(recip, expand, preferred_element_type=_F32)
    out_ref[...] = r_ref[...] * scale + b_ref[...]


def _epilogue(rst, den, bias2d):
    BR = 1000
    return pl.pallas_call(
        _epi_body,
        grid=(N // BR,),
        in_specs=[
            pl.BlockSpec((BR, HF), lambda i: (i, 0)),
            pl.BlockSpec((BR, 16), lambda i: (i, 0)),
            pl.BlockSpec((1, HF), lambda i: (0, 0)),
        ],
        out_specs=pl.BlockSpec((BR, HF), lambda i: (i, 0)),
        out_shape=jax.ShapeDtypeStruct((N, HF), _F32),
    )(rst, den, bias2d)


def kernel(x, edge_index, W, attn_l, attn_r, bias):
    Al = attn_l.reshape(H, F)
    Ar = attn_r.reshape(H, F)
    A = jnp.zeros((H, F, 16), _F32)
    hh = jnp.arange(H)[:, None]
    ff = jnp.arange(F)[None, :]
    A = A.at[hh, ff, hh].set(Al)
    A = A.at[hh, ff, hh + H].set(Ar)
    A = A.reshape(HF, 16)

    feat, elr = _prologue(x, W, A)

    pk = edge_index[0] | (edge_index[1] << 14)
    rst, den = _sc_main(elr, pk, feat)

    return _epilogue(rst, den, bias.reshape(1, HF))
